# Initial kernel scaffold; baseline (speedup 1.0000x reference)
import jax
import jax.numpy as jnp
from jax.experimental import pallas as pl


def kernel(x, embed_weight):
    raise NotImplementedError("write your pallas kernel here")


# speedup vs baseline: 6.0542x; 6.0542x over previous
import functools

import jax
import jax.numpy as jnp
from jax import lax
from jax.experimental import pallas as pl
from jax.experimental.pallas import tpu as pltpu
from jax.experimental.pallas import tpu_sc as plsc

MAX_LEN = 200
DIM = 64
ROW_WORDS = MAX_LEN * DIM
NUM_CORES = 2
NUM_SUBCORES = 16
NUM_WORKERS = NUM_CORES * NUM_SUBCORES
REP = 8
FIRE = 8
LANES = 16


@functools.partial(jax.jit, static_argnums=(1,))
def _sc_broadcast(w_flat, batch):
    rows_per_w = batch // NUM_WORKERS
    chunk_words = REP * ROW_WORDS
    chunks = rows_per_w // REP
    total = batch * ROW_WORDS

    mesh = plsc.VectorSubcoreMesh(
        core_axis_name="c", subcore_axis_name="s",
        num_cores=NUM_CORES, num_subcores=NUM_SUBCORES)

    @functools.partial(
        pl.kernel,
        mesh=mesh,
        out_type=jax.ShapeDtypeStruct((total,), jnp.float32),
        scratch_types=[
            pltpu.VMEM((ROW_WORDS,), jnp.float32),
            pltpu.VMEM((chunk_words,), jnp.float32),
            pltpu.SemaphoreType.DMA,
        ],
    )
    def k(w_hbm, out_hbm, w_v, buf_v, sem):
        wid = lax.axis_index("s") * NUM_CORES + lax.axis_index("c")
        pltpu.sync_copy(w_hbm, w_v)

        def relu_rep(i, carry):
            v = jnp.maximum(w_v[pl.ds(i * LANES, LANES)], 0.0)
            for r in range(REP):
                buf_v[pl.ds(r * ROW_WORDS + i * LANES, LANES)] = v
            return carry

        lax.fori_loop(0, ROW_WORDS // LANES, relu_rep, 0)

        base = wid * (rows_per_w * ROW_WORDS)

        def fire_drain(j, carry):
            copies = [
                pltpu.async_copy(
                    buf_v,
                    out_hbm.at[pl.ds(base + (j * FIRE + t) * chunk_words,
                                     chunk_words)],
                    sem)
                for t in range(FIRE)
            ]
            for c in copies:
                c.wait()
            return carry

        lax.fori_loop(0, chunks // FIRE, fire_drain, 0)

    return k(w_flat)


def kernel(x, embed_weight):
    batch, seq = x.shape[0], x.shape[1]
    w_flat = embed_weight[:seq].reshape(-1)
    out = _sc_broadcast(w_flat, batch)
    return out.reshape(batch, seq, DIM)


# speedup vs baseline: 6.0598x; 1.0009x over previous
import functools

import jax
import jax.numpy as jnp
from jax import lax
from jax.experimental import pallas as pl
from jax.experimental.pallas import tpu as pltpu
from jax.experimental.pallas import tpu_sc as plsc

MAX_LEN = 200
DIM = 64
ROW_WORDS = MAX_LEN * DIM
NUM_CORES = 2
NUM_SUBCORES = 16
NUM_WORKERS = NUM_CORES * NUM_SUBCORES
REP = 8
FIRE = 8
LANES = 16


@functools.partial(jax.jit, static_argnums=(1,))
def _sc_broadcast(w_flat, batch):
    rows_per_w = batch // NUM_WORKERS
    chunk_words = REP * ROW_WORDS
    chunks = rows_per_w // REP
    total = batch * ROW_WORDS

    mesh = plsc.VectorSubcoreMesh(
        core_axis_name="c", subcore_axis_name="s",
        num_cores=NUM_CORES, num_subcores=NUM_SUBCORES)

    @functools.partial(
        pl.kernel,
        mesh=mesh,
        out_type=jax.ShapeDtypeStruct((total,), jnp.float32),
        scratch_types=[
            pltpu.VMEM((ROW_WORDS,), jnp.float32),
            pltpu.VMEM((chunk_words,), jnp.float32),
            pltpu.SemaphoreType.DMA,
        ],
    )
    def k(w_hbm, out_hbm, w_v, buf_v, sem):
        wid = lax.axis_index("s") * NUM_CORES + lax.axis_index("c")
        pltpu.sync_copy(w_hbm, w_v)

        def relu_rep(i, carry):
            v = jnp.maximum(w_v[pl.ds(i * LANES, LANES)], 0.0)
            for r in range(REP):
                buf_v[pl.ds(r * ROW_WORDS + i * LANES, LANES)] = v
            return carry

        lax.fori_loop(0, ROW_WORDS // LANES, relu_rep, 0)

        base = wid * (rows_per_w * ROW_WORDS)

        def ring(j, carry):
            pltpu.async_copy(
                buf_v,
                out_hbm.at[pl.ds(base + j * chunk_words, chunk_words)],
                sem)

            @pl.when(j >= FIRE)
            def _():
                pltpu.make_async_copy(
                    buf_v, out_hbm.at[pl.ds(base, chunk_words)], sem).wait()

            return carry

        lax.fori_loop(0, chunks, ring, 0)

        def drain(j, carry):
            pltpu.make_async_copy(
                buf_v, out_hbm.at[pl.ds(base, chunk_words)], sem).wait()
            return carry

        lax.fori_loop(0, FIRE, drain, 0)

    return k(w_flat)


def kernel(x, embed_weight):
    batch, seq = x.shape[0], x.shape[1]
    w_flat = embed_weight[:seq].reshape(-1)
    out = _sc_broadcast(w_flat, batch)
    return out.reshape(batch, seq, DIM)
